# baseline (speedup 1.0000x reference)
import functools

import jax
import jax.numpy as jnp
from jax import lax
from jax.experimental import pallas as pl
from jax.experimental.pallas import tpu as pltpu
from jax.experimental.pallas import tpu_sc as plsc

_K = 8192
_D = 32
_BN = 128

_SC_NC = 2
_SC_NS = 16
_SC_NW = _SC_NC * _SC_NS


def _vq_block(x_ref, x2_ref, w2x_ref, w2_ref, lane_ref, enc_ref, idx_ref):
    x_blk = x_ref[...]
    w2x = w2x_ref[...]
    scores2 = jax.lax.dot_general(
        x_blk, w2x, (((1,), (1,)), ((), ())),
        preferred_element_type=jnp.float32)
    d2 = x2_ref[...] - scores2 + w2_ref[...]
    dist = jnp.sqrt(jnp.maximum(d2, 0.0))
    m = jnp.min(dist, axis=1, keepdims=True)
    lane = lane_ref[...]
    z = jnp.where(dist == m, lane, jnp.float32(_K))
    idx = jnp.min(z, axis=1)
    enc = jnp.where(z == idx[:, None], jnp.float32(1.0), jnp.float32(0.0))
    enc_ref[...] = enc
    idx_ref[...] = idx.astype(jnp.int32).reshape(1, 1, _BN)


def _sc_gather(w_hbm, idx_hbm, out_hbm, idx_v, rows_v, sem):
    bpw = out_hbm.shape[0] // _SC_NW
    wid = lax.axis_index("s") * _SC_NC + lax.axis_index("c")
    base = wid * bpw
    pltpu.sync_copy(idx_hbm.at[pl.ds(base, bpw)], idx_v)
    pltpu.async_copy(w_hbm.at[idx_v], rows_v, sem).wait()
    pltpu.sync_copy(rows_v, out_hbm.at[pl.ds(base, bpw)])


def kernel(x, weight):
    b, c, h, w_sp = x.shape
    x_flat = jnp.transpose(x, (0, 2, 3, 1)).reshape(-1, _D)
    n = x_flat.shape[0]
    x2 = jnp.sum(x_flat * x_flat, axis=1, keepdims=True)
    w2 = jnp.sum(weight * weight, axis=1)[None, :]
    w2x = weight + weight
    lane_row = jnp.arange(_K, dtype=jnp.float32)[None, :]
    grid = n // _BN
    enc, idx3 = pl.pallas_call(
        _vq_block,
        grid=(grid,),
        in_specs=[
            pl.BlockSpec((_BN, _D), lambda i: (i, 0)),
            pl.BlockSpec((_BN, 1), lambda i: (i, 0)),
            pl.BlockSpec((_K, _D), lambda i: (0, 0)),
            pl.BlockSpec((1, _K), lambda i: (0, 0)),
            pl.BlockSpec((1, _K), lambda i: (0, 0)),
        ],
        out_specs=[
            pl.BlockSpec((_BN, _K), lambda i: (i, 0)),
            pl.BlockSpec((1, 1, _BN), lambda i: (i, 0, 0)),
        ],
        out_shape=[
            jax.ShapeDtypeStruct((n, _K), jnp.float32),
            jax.ShapeDtypeStruct((grid, 1, _BN), jnp.int32),
        ],
    )(x_flat, x2, w2x, w2, lane_row)
    idx_flat = idx3.reshape(n)
    bpw = n // _SC_NW
    sc_fn = functools.partial(
        pl.kernel,
        mesh=plsc.VectorSubcoreMesh(core_axis_name="c", subcore_axis_name="s"),
        compiler_params=pltpu.CompilerParams(use_tc_tiling_on_sc=False),
        out_type=jax.ShapeDtypeStruct((n, _D), jnp.float32),
        scratch_types=[
            pltpu.VMEM((bpw,), jnp.int32),
            pltpu.VMEM((bpw, _D), jnp.float32),
            pltpu.SemaphoreType.DMA,
        ],
    )(_sc_gather)
    q = sc_fn(weight, idx_flat)
    quantized = jnp.transpose(q.reshape(b, h, w_sp, c), (0, 3, 1, 2))
    return enc, quantized


# speedup vs baseline: 1.5069x; 1.5069x over previous
import jax
import jax.numpy as jnp
from jax.experimental import pallas as pl

_K = 8192
_D = 32
_BN = 256


def _vq_block(x_ref, x2_ref, w2x_ref, w2_ref, lane_ref, enc_ref, q_ref):
    x_blk = x_ref[...]
    w2x = w2x_ref[...]
    scores2 = jax.lax.dot_general(
        x_blk, w2x, (((1,), (1,)), ((), ())),
        preferred_element_type=jnp.float32)
    d2 = x2_ref[...] - scores2 + w2_ref[...]
    dist = jnp.sqrt(jnp.maximum(d2, 0.0))
    m = jnp.min(dist, axis=1, keepdims=True)
    lane = lane_ref[...]
    z = jnp.where(dist == m, lane, jnp.float32(_K))
    idx = jnp.min(z, axis=1)
    enc = jnp.where(z == idx[:, None], jnp.float32(1.0), jnp.float32(0.0))
    enc_ref[...] = enc
    q_ref[...] = 0.5 * jnp.dot(enc, w2x, preferred_element_type=jnp.float32)


def kernel(x, weight):
    b, c, h, w_sp = x.shape
    x_flat = jnp.transpose(x, (0, 2, 3, 1)).reshape(-1, _D)
    n = x_flat.shape[0]
    x2 = jnp.sum(x_flat * x_flat, axis=1, keepdims=True)
    w2 = jnp.sum(weight * weight, axis=1)[None, :]
    w2x = weight + weight
    lane_row = jnp.arange(_K, dtype=jnp.float32)[None, :]
    grid = n // _BN
    enc, q = pl.pallas_call(
        _vq_block,
        grid=(grid,),
        in_specs=[
            pl.BlockSpec((_BN, _D), lambda i: (i, 0)),
            pl.BlockSpec((_BN, 1), lambda i: (i, 0)),
            pl.BlockSpec((_K, _D), lambda i: (0, 0)),
            pl.BlockSpec((1, _K), lambda i: (0, 0)),
            pl.BlockSpec((1, _K), lambda i: (0, 0)),
        ],
        out_specs=[
            pl.BlockSpec((_BN, _K), lambda i: (i, 0)),
            pl.BlockSpec((_BN, _D), lambda i: (i, 0)),
        ],
        out_shape=[
            jax.ShapeDtypeStruct((n, _K), jnp.float32),
            jax.ShapeDtypeStruct((n, _D), jnp.float32),
        ],
    )(x_flat, x2, w2x, w2, lane_row)
    quantized = jnp.transpose(q.reshape(b, h, w_sp, c), (0, 3, 1, 2))
    return enc, quantized
